# baseline (speedup 1.0000x reference)
import functools

import jax
import jax.numpy as jnp
from jax import lax
from jax.experimental import pallas as pl
from jax.experimental.pallas import tpu as pltpu
from jax.experimental.pallas import tpu_sc as plsc

_E = 16
_CAP_FACTOR = 1.25
_NW = 16


def _logits_block(x_ref, w_ref, o_ref):
    o_ref[...] = jax.lax.dot_general(
        x_ref[...], w_ref[...], (((1,), (1,)), ((), ())),
        preferred_element_type=jnp.float32)


_GDN = jax.lax.GatherDimensionNumbers(
    offset_dims=(), collapsed_slice_dims=(0,), start_index_map=(0,))


def _gather16(v, p):
    return jax.lax.gather(
        v, p[:, None], _GDN, (1,),
        mode=jax.lax.GatherScatterMode.PROMISE_IN_BOUNDS)


def _mk_butterfly(idx):
    perms = [(idx + k) & (_E - 1) for k in (1, 2, 4, 8)]

    def red(v, op):
        for p in perms:
            v = op(v, _gather16(v, p))
        return v

    return red


def _sc_router(logits_hbm, cw0_hbm, cw1_hbm, scal_hbm,
               lbuf, pos0, pos1, cw0b, cw1b, statb, allb, shared,
               *, ch, n_tokens, capacity):
    wid = lax.axis_index("s")
    base = wid * ch
    idx = lax.iota(jnp.int32, _E)
    red = _mk_butterfly(idx)

    pltpu.sync_copy(logits_hbm.at[pl.ds(base, ch)], lbuf)

    def body_a(t, carry):
        c1, c2, gs, zv = carry
        l = lbuf[t]
        zv = zv + l * l
        m = red(l, jnp.maximum)
        e = jnp.exp(l - m)
        s = red(e, jnp.add)
        i1 = red(jnp.where(l == m, idx, _E), jnp.minimum)
        oh1 = idx == i1
        mk = jnp.where(oh1, -jnp.float32(3.4e38), l)
        m2 = red(mk, jnp.maximum)
        i2 = red(jnp.where(mk == m2, idx, _E), jnp.minimum)
        oh2 = idx == i2
        v1 = 1.0 / s
        v2 = jnp.exp(m2 - m) / s
        oh1f = jnp.where(oh1, 1.0, 0.0)
        oh2f = jnp.where(oh2, 1.0, 0.0)
        pos0[t] = jnp.where(oh1, c1, 1e9)
        pos1[t] = jnp.where(oh2, c2, 1e9)
        cw0b[t] = oh1f * v1
        cw1b[t] = oh2f * jnp.maximum(v2, 1e-30)
        return (c1 + oh1f, c2 + oh2f, gs + e / s, zv)

    zeros = jnp.zeros((_E,), jnp.float32)
    c1, c2, gs, zv = lax.fori_loop(0, ch, body_a, (zeros, zeros, zeros, zeros))

    statb[0] = c1
    statb[1] = c2
    statb[2] = gs
    statb[3] = zv
    pltpu.sync_copy(statb, shared.at[pl.ds(wid * 4, 4)])
    plsc.subcore_barrier()
    pltpu.sync_copy(shared, allb)

    def body_off(w, carry):
        o1, o2, t1, gt, zt = carry
        take = jnp.where(w < wid, 1.0, 0.0)
        r1 = allb[w * 4]
        r2 = allb[w * 4 + 1]
        o1 = o1 + r1 * take
        o2 = o2 + r2 * take
        return (o1, o2, t1 + r1, gt + allb[w * 4 + 2], zt + allb[w * 4 + 3])

    o1, o2, t1, gt, zt = lax.fori_loop(
        0, _NW, body_off, (zeros, zeros, zeros, zeros, zeros))

    cap = jnp.float32(capacity)

    def body_b(t, _):
        k0 = (pos0[t] + o1) < cap
        k1 = (pos1[t] + o2) < cap
        cw0b[t] = jnp.where(k0, cw0b[t], 0.0)
        cw1b[t] = jnp.where(k1, cw1b[t], 0.0)
        return 0

    lax.fori_loop(0, ch, body_b, 0)

    pltpu.sync_copy(cw0b, cw0_hbm.at[pl.ds(base, ch)])
    pltpu.sync_copy(cw1b, cw1_hbm.at[pl.ds(base, ch)])

    n_f = jnp.float32(n_tokens)
    aux = _E * red((gt / n_f) * (t1 / n_f), jnp.add)
    z = red(zt, jnp.add) / (n_f * _E)
    statb[0] = jnp.where(idx == 0, aux, jnp.where(idx == 1, z, 0.0))

    @pl.when(wid == 0)
    def _emit_scalars():
        pltpu.sync_copy(statb.at[0], scal_hbm)


def kernel(x, W):
    B, T, C = x.shape
    N = B * T
    E = W.shape[0]
    capacity = int(_CAP_FACTOR * N * 2 / E)
    blk = 1024
    ch = N // _NW

    x2 = x.reshape(N, C)

    logits = pl.pallas_call(
        _logits_block,
        grid=(N // blk,),
        in_specs=[
            pl.BlockSpec((blk, C), lambda i: (i, 0)),
            pl.BlockSpec((E, C), lambda i: (0, 0)),
        ],
        out_specs=pl.BlockSpec((blk, E), lambda i: (i, 0)),
        out_shape=jax.ShapeDtypeStruct((N, E), jnp.float32),
    )(x2, W)

    mesh = plsc.VectorSubcoreMesh(
        core_axis_name="c", subcore_axis_name="s", num_cores=1)
    router = functools.partial(
        pl.kernel,
        mesh=mesh,
        compiler_params=pltpu.CompilerParams(use_tc_tiling_on_sc=False),
        out_type=(
            jax.ShapeDtypeStruct((N, E), jnp.float32),
            jax.ShapeDtypeStruct((N, E), jnp.float32),
            jax.ShapeDtypeStruct((E,), jnp.float32),
        ),
        scratch_types=[
            pltpu.VMEM((ch, E), jnp.float32),
            pltpu.VMEM((ch, E), jnp.float32),
            pltpu.VMEM((ch, E), jnp.float32),
            pltpu.VMEM((ch, E), jnp.float32),
            pltpu.VMEM((ch, E), jnp.float32),
            pltpu.VMEM((4, E), jnp.float32),
            pltpu.VMEM((4 * _NW, E), jnp.float32),
            pltpu.VMEM_SHARED((4 * _NW, E), jnp.float32),
        ],
    )(functools.partial(_sc_router, ch=ch, n_tokens=N, capacity=capacity))
    cw0, cw1, scal = router(logits)

    combine_weights = jnp.stack([cw0, cw1], axis=-1)
    dispatch_mask = combine_weights > 0.0
    return (dispatch_mask, combine_weights, scal[0], scal[1])


# speedup vs baseline: 2.0178x; 2.0178x over previous
import functools

import jax
import jax.numpy as jnp
from jax.experimental import pallas as pl
from jax.experimental.pallas import tpu as pltpu

_E = 16
_CAP_FACTOR = 1.25


def _router_block(x_ref, w2_ref, cw_ref, scal_ref, acc_ref,
                  *, blk, n_tokens, capacity, nblk):
    i = pl.program_id(0)
    e2 = 2 * _E

    @pl.when(i == 0)
    def _init():
        acc_ref[...] = jnp.zeros_like(acc_ref)

    xb = x_ref[...]
    w2 = w2_ref[...]
    logits2 = jax.lax.dot_general(
        w2, xb, (((1,), (1,)), ((), ())), preferred_element_type=jnp.float32)

    zpart = jnp.sum(logits2 * logits2) * 0.5

    m = jnp.max(logits2, axis=0, keepdims=True)
    eg = jnp.exp(logits2 - m)
    s2 = jnp.sum(eg, axis=0, keepdims=True)
    gates2 = eg * (2.0 / s2)

    rowid = jax.lax.broadcasted_iota(jnp.int32, (e2, blk), 0)
    is_even = rowid % 2 == 0

    v1 = jnp.max(gates2, axis=0, keepdims=True)
    r1 = jnp.min(jnp.where((gates2 == v1) & is_even, rowid, e2),
                 axis=0, keepdims=True)
    same_e = (rowid // 2) == (r1 // 2)
    masked = jnp.where(same_e, -jnp.inf, gates2)
    v2 = jnp.max(masked, axis=0, keepdims=True)
    r2 = jnp.min(jnp.where((masked == v2) & (~is_even), rowid, e2),
                 axis=0, keepdims=True)

    oh = (rowid == r1) | (rowid == r2)
    ohf = oh.astype(jnp.float32)

    rr = jax.lax.broadcasted_iota(jnp.int32, (blk, blk), 0)
    cc = jax.lax.broadcasted_iota(jnp.int32, (blk, blk), 1)
    utri = (rr <= cc).astype(jnp.bfloat16)
    cum = jnp.dot(ohf.astype(jnp.bfloat16), utri,
                  preferred_element_type=jnp.float32)

    counts = acc_ref[:, 0:1]
    pos = cum - 1.0 + counts
    keep = oh & (pos < capacity)

    cw_ref[...] = keep.astype(jnp.float32) * jnp.maximum(gates2, 1e-30)

    acc_ref[:, 0:1] = counts + jnp.sum(ohf, axis=1, keepdims=True)
    acc_ref[:, 1:2] = acc_ref[:, 1:2] + jnp.sum(gates2, axis=1, keepdims=True)
    acc_ref[:, 2:3] = acc_ref[:, 2:3] + zpart

    @pl.when(i == nblk - 1)
    def _finish():
        n_f = jnp.float32(n_tokens)
        cnt = acc_ref[:, 0:1]
        gsum = acc_ref[:, 1:2]
        col = jax.lax.broadcasted_iota(jnp.int32, (e2, 1), 0)
        auxsum = jnp.sum(jnp.where(col % 2 == 0, gsum * cnt, 0.0))
        aux = _E * auxsum / (n_f * n_f)
        z = jnp.max(acc_ref[:, 2:3]) / (n_f * _E)
        lane = jax.lax.broadcasted_iota(jnp.int32, (1, _E), 1)
        scal_ref[...] = jnp.where(lane == 0, aux, jnp.where(lane == 1, z, 0.0))


def kernel(x, W):
    B, T, C = x.shape
    N = B * T
    E = W.shape[0]
    capacity = int(_CAP_FACTOR * N * 2 / E)
    blk = 1024
    nblk = N // blk

    x2 = x.reshape(N, C)
    w2 = jnp.repeat(W, 2, axis=0)

    body = functools.partial(
        _router_block, blk=blk, n_tokens=N, capacity=capacity, nblk=nblk)

    out_shapes = (
        jax.ShapeDtypeStruct((2 * E, N), jnp.float32),
        jax.ShapeDtypeStruct((1, E), jnp.float32),
    )
    grid = (nblk,)
    cw, scal = pl.pallas_call(
        body,
        grid=grid,
        in_specs=[
            pl.BlockSpec((blk, C), lambda i: (i, 0)),
            pl.BlockSpec((2 * E, C), lambda i: (0, 0)),
        ],
        out_specs=[
            pl.BlockSpec((2 * E, blk), lambda i: (0, i)),
            pl.BlockSpec((1, E), lambda i: (0, 0)),
        ],
        out_shape=out_shapes,
        scratch_shapes=[pltpu.VMEM((2 * E, 128), jnp.float32)],
    )(x2, w2)

    cwt = cw.T.reshape(N, E, 2)
    dispatch_mask = cwt > 0.0
    combine_weights = cwt
    aux_loss = scal[0, 0]
    z_loss = scal[0, 1]
    return (dispatch_mask, combine_weights, aux_loss, z_loss)
